# baseline (speedup 1.0000x reference)
import functools

import jax
import jax.numpy as jnp
from jax.experimental import pallas as pl
from jax.experimental.pallas import tpu as pltpu

_P = 8


def _fold3_body(s_ref, w_ref, o_ref):
    s = s_ref[...]
    w = w_ref[...]
    o_ref[...] = (s[0][:, None] * w[0] + s[1][:, None] * w[1]
                  + s[2][:, None] * w[2])


def _fold3(s3, w3, row_tile):
    c = w3.shape[1]
    return pl.pallas_call(
        _fold3_body,
        grid=(c // row_tile,),
        in_specs=[
            pl.BlockSpec((3, row_tile), lambda g: (0, g)),
            pl.BlockSpec((3, row_tile, 64), lambda g: (0, g, 0)),
        ],
        out_specs=pl.BlockSpec((row_tile, 64), lambda g: (g, 0)),
        out_shape=jax.ShapeDtypeStruct((c, 64), jnp.float32),
    )(s3, w3)


def _imap(j, g, idx_ref):
    return (idx_ref[g * _P + j], 0, 0)


def _timap(j, g, idx_ref):
    return (idx_ref[g * 3 * _P + j], 0, 0)


def _level_body(idx_ref, *refs):
    tbl = refs[:_P]
    s_ref, w_ref, out_ref, a_ref = refs[_P:]

    @pl.when(pl.program_id(0) == 0)
    def _():
        s = s_ref[...]
        w = w_ref[...]
        a_ref[...] = (s[0][:, None] * w[0] + s[1][:, None] * w[1]
                      + s[2][:, None] * w[2])

    rows = jnp.concatenate([r[0] for r in tbl], axis=0)
    r = jnp.maximum(rows, 0.0)
    acc = jnp.dot(r, a_ref[...], preferred_element_type=jnp.float32)
    ss = jnp.sum(r * r, axis=1, keepdims=True)
    out_ref[...] = jnp.concatenate(
        [acc, jnp.broadcast_to(ss, (_P, 64))], axis=1)


def _level3_body(idx_ref, *refs):
    tbl = refs[:_P]
    a_ref, out_ref = refs[_P:]
    rows = jnp.concatenate([r[0] for r in tbl], axis=0)
    r = jnp.maximum(rows, 0.0)
    acc = jnp.dot(r, a_ref[...], preferred_element_type=jnp.float32)
    ss = jnp.sum(r * r, axis=1, keepdims=True)
    out_ref[...] = jnp.concatenate(
        [acc, jnp.broadcast_to(ss, (_P, 64))], axis=1)


def _timet_body(idx_ref, *refs):
    tbl = refs[:3 * _P]
    st_ref, wt_ref, out_ref = refs[3 * _P:]
    rows = jnp.concatenate(
        [jnp.concatenate(
            [tbl[3 * i][0], tbl[3 * i + 1][0], tbl[3 * i + 2][0]], axis=1)
         for i in range(_P)], axis=0)
    r = jnp.maximum(rows, 0.0)
    acc = jnp.dot(r * st_ref[...], wt_ref[...],
                  preferred_element_type=jnp.float32)
    ss = jnp.sum(r * r, axis=1, keepdims=True)
    out_ref[...] = jnp.concatenate(
        [acc, jnp.broadcast_to(ss, (_P, 64))], axis=1)


def _combine_body(d_feat, p0, p1, p2, p3, pt, tail_ref, wf_ref, bf_ref,
                  out_ref):
    acc = (p0[:, :64] + p1[:, :64] + p2[:, :64] + p3[:, :64] + pt[:, :64])
    ss = 3.0 * (p0[:, 64:65] + p1[:, 64:65] + p2[:, 64:65] + p3[:, 64:65]) \
        + pt[:, 64:65]
    denom = jnp.sqrt(ss) * (float(d_feat) ** -0.5) + 1e-8
    d = acc / denom
    wf = wf_ref[...]
    out_ref[...] = (jnp.dot(d, wf[:64], preferred_element_type=jnp.float32)
                    + jnp.dot(tail_ref[...], wf[64:],
                              preferred_element_type=jnp.float32)
                    + bf_ref[...])


def _gather_level(idx, table3d, s, w, b):
    c = table3d.shape[2]
    grid_spec = pltpu.PrefetchScalarGridSpec(
        num_scalar_prefetch=1,
        grid=(b // _P,),
        in_specs=(
            [pl.BlockSpec((1, 1, c), functools.partial(_imap, j))
             for j in range(_P)]
            + [pl.BlockSpec((3, c), lambda g, i: (0, 0)),
               pl.BlockSpec((3, c, 64), lambda g, i: (0, 0, 0))]),
        out_specs=pl.BlockSpec((_P, 128), lambda g, i: (g, 0)),
        scratch_shapes=[pltpu.VMEM((c, 64), jnp.float32)],
    )
    return pl.pallas_call(
        _level_body, grid_spec=grid_spec,
        out_shape=jax.ShapeDtypeStruct((b, 128), jnp.float32),
    )(idx, *([table3d] * _P), s, w)


def _gather_level3(idx, table3d, a3, b):
    c = table3d.shape[2]
    grid_spec = pltpu.PrefetchScalarGridSpec(
        num_scalar_prefetch=1,
        grid=(b // _P,),
        in_specs=(
            [pl.BlockSpec((1, 1, c), functools.partial(_imap, j))
             for j in range(_P)]
            + [pl.BlockSpec((c, 64), lambda g, i: (0, 0))]),
        out_specs=pl.BlockSpec((_P, 128), lambda g, i: (g, 0)),
    )
    return pl.pallas_call(
        _level3_body, grid_spec=grid_spec,
        out_shape=jax.ShapeDtypeStruct((b, 128), jnp.float32),
    )(idx, *([table3d] * _P), a3)


def _gather_time(idx, table3d, st, wt, b):
    c = table3d.shape[2]
    grid_spec = pltpu.PrefetchScalarGridSpec(
        num_scalar_prefetch=1,
        grid=(b // _P,),
        in_specs=(
            [pl.BlockSpec((1, 1, c), functools.partial(_timap, j))
             for j in range(3 * _P)]
            + [pl.BlockSpec((1, 3 * c), lambda g, i: (0, 0)),
               pl.BlockSpec((3 * c, 64), lambda g, i: (0, 0))]),
        out_specs=pl.BlockSpec((_P, 128), lambda g, i: (g, 0)),
    )
    return pl.pallas_call(
        _timet_body, grid_spec=grid_spec,
        out_shape=jax.ShapeDtypeStruct((b, 128), jnp.float32),
    )(idx, *([table3d] * (3 * _P)), st, wt)


def kernel(pos, dir, t, table0, table1, table2, table3, ts_table1, rms_scale,
           W_seq, W_fin, b_fin):
    b = pos.shape[0]
    d_feat = rms_scale.shape[0]

    def flat_idx(res, tdim):
        idx = (pos * (res - 1)).astype(jnp.int32)
        return ((jnp.mod(idx[:, 0] - 1, res) * res
                 + jnp.mod(idx[:, 1], res)) * res
                + jnp.mod(idx[:, 2] + 1, res))

    i0 = flat_idx(128, None)
    i1 = flat_idx(64, None)
    i2 = flat_idx(32, None)
    i3 = flat_idx(16, None)
    idx3u = (pos * 15).astype(jnp.int32)
    sp3 = ((jnp.mod(idx3u[:, 0], 16) * 16
            + jnp.mod(idx3u[:, 1], 16)) * 16
           + jnp.mod(idx3u[:, 2], 16))
    t_idx = (t * 127).astype(jnp.int32)
    it = jnp.stack([sp3 * 64 + jnp.mod(t_idx + k, 64) for k in (-1, 0, 1)],
                   axis=1).reshape(3 * b)

    c0, c1, c2, c3 = (table0.shape[3], table1.shape[3], table2.shape[3],
                      table3.shape[3])
    o1 = 3 * c0
    o2 = o1 + 3 * c1
    o3 = o2 + 3 * c2
    ot = o3 + 3 * c3
    w0 = W_seq[:o1].reshape(3, c0, 64)
    s0 = rms_scale[:o1].reshape(3, c0)
    w1 = W_seq[o1:o2].reshape(3, c1, 64)
    s1 = rms_scale[o1:o2].reshape(3, c1)
    w2 = W_seq[o2:o3].reshape(3, c2, 64)
    s2 = rms_scale[o2:o3].reshape(3, c2)
    w3 = W_seq[o3:ot].reshape(3, c3, 64)
    s3 = rms_scale[o3:ot].reshape(3, c3)
    wt = W_seq[ot:]
    st = rms_scale[ot:].reshape(1, 192)

    t0 = table0.reshape(-1, 1, c0)
    t1 = table1.reshape(-1, 1, c1)
    t2 = table2.reshape(-1, 1, c2)
    t3 = table3.reshape(-1, 1, c3)
    tt = ts_table1.reshape(-1, 1, 64)

    a3 = _fold3(s3, w3, 2048)

    p0 = _gather_level(i0, t0, s0, w0, b)
    p1 = _gather_level(i1, t1, s1, w1, b)
    p2 = _gather_level(i2, t2, s2, w2, b)
    p3 = _gather_level3(i3, t3, a3, b)
    pt = _gather_time(it, tt, st, wt, b)

    tail = jnp.concatenate([dir, t[:, None]], axis=1)
    out = pl.pallas_call(
        functools.partial(_combine_body, d_feat),
        grid=(1,),
        in_specs=[pl.BlockSpec((b, 128), lambda g: (0, 0))] * 5
        + [pl.BlockSpec((b, 4), lambda g: (0, 0)),
           pl.BlockSpec((68, 4), lambda g: (0, 0)),
           pl.BlockSpec((1, 4), lambda g: (0, 0))],
        out_specs=pl.BlockSpec((b, 4), lambda g: (0, 0)),
        out_shape=jax.ShapeDtypeStruct((b, 4), jnp.float32),
    )(p0, p1, p2, p3, pt, tail, W_fin, b_fin.reshape(1, 4))
    return out


# speedup vs baseline: 3.8737x; 3.8737x over previous
import functools

import jax
import jax.numpy as jnp
from jax import lax
from jax.experimental import pallas as pl
from jax.experimental.pallas import tpu as pltpu
from jax.experimental.pallas import tpu_sc as plsc

_P = 8


def _fold3_body(s_ref, w_ref, o_ref):
    s = s_ref[...]
    w = w_ref[...]
    o_ref[...] = (s[0][:, None] * w[0] + s[1][:, None] * w[1]
                  + s[2][:, None] * w[2])


def _fold3(s3, w3):
    c = w3.shape[1]
    row_tile = min(c, 2048)
    return pl.pallas_call(
        _fold3_body,
        grid=(c // row_tile,),
        in_specs=[
            pl.BlockSpec((3, row_tile), lambda g: (0, g)),
            pl.BlockSpec((3, row_tile, 64), lambda g: (0, g, 0)),
        ],
        out_specs=pl.BlockSpec((row_tile, 64), lambda g: (g, 0)),
        out_shape=jax.ShapeDtypeStruct((c, 64), jnp.float32),
    )(s3, w3)


def _sc_gather(i0, i1, i2, it, t0, t1, t2, tt):
    b = i0.shape[0]
    info = plsc.get_sparse_core_info()
    nc, ns = info.num_cores, info.num_subcores
    nw = nc * ns
    bpw = b // nw
    tpw = (3 * b) // nw
    c0, c1, c2, ct = t0.shape[1], t1.shape[1], t2.shape[1], tt.shape[1]
    l2_chunk = 8
    f32 = jnp.float32

    def body(i0_h, i1_h, i2_h, it_h, t0_h, t1_h, t2_h, tt_h,
             o0_h, o1_h, o2_h, ot_h,
             idx_v, idxt_v, r0_v, r1_v, r2_v, rt_v, sem):
        wid = lax.axis_index("s") * nc + lax.axis_index("c")
        base = wid * bpw
        baset = wid * tpw
        pltpu.sync_copy(i0_h.at[pl.ds(base, bpw)], idx_v)
        pltpu.async_copy(t0_h.at[idx_v], r0_v, sem).wait()
        pltpu.sync_copy(r0_v, o0_h.at[pl.ds(base, bpw)])
        pltpu.sync_copy(i1_h.at[pl.ds(base, bpw)], idx_v)
        pltpu.async_copy(t1_h.at[idx_v], r1_v, sem).wait()
        pltpu.sync_copy(r1_v, o1_h.at[pl.ds(base, bpw)])
        pltpu.sync_copy(i2_h.at[pl.ds(base, bpw)], idx_v)
        for ch in range(bpw // l2_chunk):
            pltpu.async_copy(
                t2_h.at[idx_v.at[pl.ds(ch * l2_chunk, l2_chunk)]], r2_v,
                sem).wait()
            pltpu.sync_copy(
                r2_v, o2_h.at[pl.ds(base + ch * l2_chunk, l2_chunk)])
        pltpu.sync_copy(it_h.at[pl.ds(baset, tpw)], idxt_v)
        pltpu.async_copy(tt_h.at[idxt_v], rt_v, sem).wait()
        pltpu.sync_copy(rt_v, ot_h.at[pl.ds(baset, tpw)])

    out_type = (jax.ShapeDtypeStruct((b, c0), f32),
                jax.ShapeDtypeStruct((b, c1), f32),
                jax.ShapeDtypeStruct((b, c2), f32),
                jax.ShapeDtypeStruct((3 * b, ct), f32))
    mesh = plsc.VectorSubcoreMesh(core_axis_name="c", subcore_axis_name="s")
    return pl.kernel(
        body, out_type=out_type, mesh=mesh,
        scratch_types=[pltpu.VMEM((bpw,), jnp.int32),
                       pltpu.VMEM((tpw,), jnp.int32),
                       pltpu.VMEM((bpw, c0), f32),
                       pltpu.VMEM((bpw, c1), f32),
                       pltpu.VMEM((l2_chunk, c2), f32),
                       pltpu.VMEM((tpw, ct), f32),
                       pltpu.SemaphoreType.DMA],
    )(i0, i1, i2, it, t0, t1, t2, tt)


def _main_body(d_feat, nsteps, i3_ref, t3_ref, g0_ref, g1_ref, g2_ref, gt_ref,
               a0_ref, a1_ref, a2_ref, a3_ref, st_ref, wt_ref, tail_ref,
               wf_ref, bf_ref, out_ref, buf3, sem):
    g = pl.program_id(0)
    slot = lax.rem(g, 2)
    nslot = 1 - slot

    def cp(step, s, j):
        return pltpu.make_async_copy(
            t3_ref.at[i3_ref[step * _P + j]], buf3.at[s, j], sem.at[s, j])

    @pl.when(g == 0)
    def _():
        for j in range(_P):
            cp(0, 0, j).start()

    @pl.when(g + 1 < nsteps)
    def _():
        for j in range(_P):
            cp(g + 1, nslot, j).start()

    for j in range(_P):
        cp(g, slot, j).wait()

    tl = tail_ref[...]
    r3 = jnp.maximum(buf3[slot], 0.0)
    acc = jnp.dot(r3, a3_ref[...], preferred_element_type=jnp.float32)
    ss_sp = jnp.sum(r3 * r3, axis=1, keepdims=True)
    w0 = g0_ref[...]
    r0 = jnp.maximum(jnp.where(tl[:, 4:5] > 0.5, w0[:, 64:], w0[:, :64]), 0.0)
    acc += jnp.dot(r0, a0_ref[...], preferred_element_type=jnp.float32)
    ss_sp += jnp.sum(r0 * r0, axis=1, keepdims=True)
    r1 = jnp.maximum(g1_ref[...], 0.0)
    acc += jnp.dot(r1, a1_ref[...], preferred_element_type=jnp.float32)
    ss_sp += jnp.sum(r1 * r1, axis=1, keepdims=True)
    r2 = jnp.maximum(g2_ref[...], 0.0)
    acc += jnp.dot(r2, a2_ref[...], preferred_element_type=jnp.float32)
    ss_sp += jnp.sum(r2 * r2, axis=1, keepdims=True)
    wt3 = gt_ref[...]
    rt = jnp.maximum(jnp.concatenate(
        [jnp.where(tl[:, 5 + k:6 + k] > 0.5,
                   wt3[:, 128 * k + 64:128 * k + 128],
                   wt3[:, 128 * k:128 * k + 64]) for k in range(3)],
        axis=1), 0.0)
    acc += jnp.dot(rt * st_ref[...], wt_ref[...],
                   preferred_element_type=jnp.float32)
    ss = 3.0 * ss_sp + jnp.sum(rt * rt, axis=1, keepdims=True)
    denom = jnp.sqrt(ss) * (float(d_feat) ** -0.5) + 1e-8
    d = acc / denom
    wf = wf_ref[...]
    out_ref[...] = (jnp.dot(d, wf[:64], preferred_element_type=jnp.float32)
                    + jnp.dot(tl[:, :4], wf[64:],
                              preferred_element_type=jnp.float32)
                    + bf_ref[...])


def kernel(pos, dir, t, table0, table1, table2, table3, ts_table1, rms_scale,
           W_seq, W_fin, b_fin):
    b = pos.shape[0]
    d_feat = rms_scale.shape[0]

    def flat_idx(res):
        idx = (pos * (res - 1)).astype(jnp.int32)
        return ((jnp.mod(idx[:, 0] - 1, res) * res
                 + jnp.mod(idx[:, 1], res)) * res
                + jnp.mod(idx[:, 2] + 1, res))

    i0 = flat_idx(128)
    i1 = flat_idx(64)
    i2 = flat_idx(32)
    i3 = flat_idx(16)
    idx3u = (pos * 15).astype(jnp.int32)
    sp3 = ((jnp.mod(idx3u[:, 0], 16) * 16
            + jnp.mod(idx3u[:, 1], 16)) * 16
           + jnp.mod(idx3u[:, 2], 16))
    t_idx = (t * 127).astype(jnp.int32)
    it = jnp.stack([sp3 * 64 + jnp.mod(t_idx + k, 64) for k in (-1, 0, 1)],
                   axis=1).reshape(3 * b)

    c0, c1, c2, c3 = (table0.shape[3], table1.shape[3], table2.shape[3],
                      table3.shape[3])
    o1 = 3 * c0
    o2 = o1 + 3 * c1
    o3 = o2 + 3 * c2
    ot = o3 + 3 * c3
    a0 = _fold3(rms_scale[:o1].reshape(3, c0), W_seq[:o1].reshape(3, c0, 64))
    a1 = _fold3(rms_scale[o1:o2].reshape(3, c1),
                W_seq[o1:o2].reshape(3, c1, 64))
    a2 = _fold3(rms_scale[o2:o3].reshape(3, c2),
                W_seq[o2:o3].reshape(3, c2, 64))
    a3 = _fold3(rms_scale[o3:ot].reshape(3, c3),
                W_seq[o3:ot].reshape(3, c3, 64))
    wt = W_seq[ot:]
    st = rms_scale[ot:].reshape(1, 192)

    g0, g1, g2, gt3 = _sc_gather(
        i0 // 2, i1, i2, it // 2,
        table0.reshape(-1, 2 * c0), table1.reshape(-1, c1),
        table2.reshape(-1, c2), ts_table1.reshape(-1, 128))
    gt = gt3.reshape(b, 384)

    tail = jnp.concatenate(
        [dir, t[:, None], (i0 % 2)[:, None].astype(jnp.float32),
         (it % 2).reshape(b, 3).astype(jnp.float32)], axis=1)
    nsteps = b // _P
    grid_spec = pltpu.PrefetchScalarGridSpec(
        num_scalar_prefetch=1,
        grid=(nsteps,),
        in_specs=[
            pl.BlockSpec(memory_space=pl.ANY),
            pl.BlockSpec((_P, 2 * c0), lambda g, i: (g, 0)),
            pl.BlockSpec((_P, c1), lambda g, i: (g, 0)),
            pl.BlockSpec((_P, c2), lambda g, i: (g, 0)),
            pl.BlockSpec((_P, 384), lambda g, i: (g, 0)),
            pl.BlockSpec((c0, 64), lambda g, i: (0, 0)),
            pl.BlockSpec((c1, 64), lambda g, i: (0, 0)),
            pl.BlockSpec((c2, 64), lambda g, i: (0, 0)),
            pl.BlockSpec((c3, 64), lambda g, i: (0, 0)),
            pl.BlockSpec((1, 192), lambda g, i: (0, 0)),
            pl.BlockSpec((192, 64), lambda g, i: (0, 0)),
            pl.BlockSpec((_P, 8), lambda g, i: (g, 0)),
            pl.BlockSpec((68, 4), lambda g, i: (0, 0)),
            pl.BlockSpec((1, 4), lambda g, i: (0, 0)),
        ],
        out_specs=pl.BlockSpec((_P, 4), lambda g, i: (g, 0)),
        scratch_shapes=[pltpu.VMEM((2, _P, c3), jnp.float32),
                        pltpu.SemaphoreType.DMA((2, _P))],
    )
    return pl.pallas_call(
        functools.partial(_main_body, d_feat, nsteps),
        grid_spec=grid_spec,
        out_shape=jax.ShapeDtypeStruct((b, 4), jnp.float32),
        compiler_params=pltpu.CompilerParams(
            dimension_semantics=("arbitrary",)),
    )(i3, table3.reshape(-1, c3), g0, g1, g2, gt, a0, a1, a2, a3, st, wt,
      tail, W_fin, b_fin.reshape(1, 4))


# speedup vs baseline: 6.7837x; 1.7512x over previous
import functools

import jax
import jax.numpy as jnp
from jax import lax
from jax.experimental import pallas as pl
from jax.experimental.pallas import tpu as pltpu
from jax.experimental.pallas import tpu_sc as plsc

_P = 8


def _fold3_body(s_ref, w_ref, o_ref):
    s = s_ref[...]
    w = w_ref[...]
    o_ref[...] = (s[0][:, None] * w[0] + s[1][:, None] * w[1]
                  + s[2][:, None] * w[2])


def _fold3(s3, w3):
    c = w3.shape[1]
    row_tile = min(c, 2048)
    return pl.pallas_call(
        _fold3_body,
        grid=(c // row_tile,),
        in_specs=[
            pl.BlockSpec((3, row_tile), lambda g: (0, g)),
            pl.BlockSpec((3, row_tile, 64), lambda g: (0, g, 0)),
        ],
        out_specs=pl.BlockSpec((row_tile, 64), lambda g: (g, 0)),
        out_shape=jax.ShapeDtypeStruct((c, 64), jnp.float32),
    )(s3, w3)


def _sc_gather(i1, i2, i3, t1, t2, t3):
    b = i1.shape[0]
    info = plsc.get_sparse_core_info()
    nc, ns = info.num_cores, info.num_subcores
    nw = nc * ns
    bpw = b // nw
    c1, c2, c3 = t1.shape[1], t2.shape[1], t3.shape[1]
    ch2 = 8
    ch3 = 8
    cq3 = c3 // 4
    f32 = jnp.float32

    def body(i1_h, i2_h, i3_h, t1_h, t2_h, t3_h, o1_h, o2_h, o3_h,
             idx_v, r1_v, r2_v, r3_v, sem):
        wid = lax.axis_index("s") * nc + lax.axis_index("c")
        base = wid * bpw
        pltpu.sync_copy(i1_h.at[pl.ds(base, bpw)], idx_v)
        pltpu.async_copy(t1_h.at[idx_v], r1_v, sem).wait()
        pltpu.sync_copy(r1_v, o1_h.at[pl.ds(base, bpw)])
        pltpu.sync_copy(i2_h.at[pl.ds(base, bpw)], idx_v)
        for ch in range(bpw // ch2):
            pltpu.async_copy(
                t2_h.at[idx_v.at[pl.ds(ch * ch2, ch2)]], r2_v, sem).wait()
            pltpu.sync_copy(r2_v, o2_h.at[pl.ds(base + ch * ch2, ch2)])
        pltpu.sync_copy(i3_h.at[pl.ds(base, bpw)], idx_v)
        for ch in range(bpw // ch3):
            for q in range(4):
                pltpu.async_copy(
                    t3_h.at[idx_v.at[pl.ds(ch * ch3, ch3)],
                            pl.ds(q * cq3, cq3)],
                    r3_v, sem).wait()
                pltpu.sync_copy(
                    r3_v,
                    o3_h.at[pl.ds(base + ch * ch3, ch3), pl.ds(q * cq3, cq3)])

    out_type = (jax.ShapeDtypeStruct((b, c1), f32),
                jax.ShapeDtypeStruct((b, c2), f32),
                jax.ShapeDtypeStruct((b, c3), f32))
    mesh = plsc.VectorSubcoreMesh(core_axis_name="c", subcore_axis_name="s")
    return pl.kernel(
        body, out_type=out_type, mesh=mesh,
        scratch_types=[pltpu.VMEM((bpw,), jnp.int32),
                       pltpu.VMEM((bpw, c1), f32),
                       pltpu.VMEM((ch2, c2), f32),
                       pltpu.VMEM((ch3, cq3), f32),
                       pltpu.SemaphoreType.DMA],
    )(i1, i2, i3, t1, t2, t3)


def _sel8(refs, rem):
    rows = jnp.stack([r[0] for r in refs])
    iota = lax.broadcasted_iota(jnp.int32, (1, 8), 1).astype(jnp.float32)
    m = jnp.where(rem == iota, 1.0, 0.0)
    return jnp.sum(rows * m[:, :, None], axis=1)


def _main_body(d_feat, *refs):
    sp_ref = refs[0]
    g1_ref, g2_ref, g3_ref = refs[1:4]
    t0b = refs[4:4 + _P]
    ttb = refs[4 + _P:4 + _P + 3 * _P]
    (a0_ref, a1_ref, a2_ref, a3_ref, st_ref, wt_ref, tail_ref, wf_ref,
     bf_ref, out_ref) = refs[4 + 4 * _P:]

    tl = tail_ref[...]
    r3 = jnp.maximum(g3_ref[...], 0.0)
    acc = jnp.dot(r3, a3_ref[...], preferred_element_type=jnp.float32)
    ss_sp = jnp.sum(r3 * r3, axis=1, keepdims=True)
    r0 = jnp.maximum(_sel8(t0b, tl[:, 4:5]), 0.0)
    acc += jnp.dot(r0, a0_ref[...], preferred_element_type=jnp.float32)
    ss_sp += jnp.sum(r0 * r0, axis=1, keepdims=True)
    r1 = jnp.maximum(g1_ref[...], 0.0)
    acc += jnp.dot(r1, a1_ref[...], preferred_element_type=jnp.float32)
    ss_sp += jnp.sum(r1 * r1, axis=1, keepdims=True)
    r2 = jnp.maximum(g2_ref[...], 0.0)
    acc += jnp.dot(r2, a2_ref[...], preferred_element_type=jnp.float32)
    ss_sp += jnp.sum(r2 * r2, axis=1, keepdims=True)
    rt = jnp.maximum(jnp.concatenate(
        [_sel8(ttb[k::3], tl[:, 5 + k:6 + k]) for k in range(3)], axis=1),
        0.0)
    acc += jnp.dot(rt * st_ref[...], wt_ref[...],
                   preferred_element_type=jnp.float32)
    ss = 3.0 * ss_sp + jnp.sum(rt * rt, axis=1, keepdims=True)
    denom = jnp.sqrt(ss) * (float(d_feat) ** -0.5) + 1e-8
    d = acc / denom
    wf = wf_ref[...]
    out_ref[...] = (jnp.dot(d, wf[:64], preferred_element_type=jnp.float32)
                    + jnp.dot(tl[:, :4], wf[64:],
                              preferred_element_type=jnp.float32)
                    + bf_ref[...])


def _t0map(j, g, sp):
    return (sp[0, g * _P + j], sp[1, g * _P + j], 0)


def _ttmap(j, g, sp):
    return (sp[2, g * _P + j // 3], sp[3 + j % 3, g * _P + j // 3], 0)


def kernel(pos, dir, t, table0, table1, table2, table3, ts_table1, rms_scale,
           W_seq, W_fin, b_fin):
    b = pos.shape[0]
    d_feat = rms_scale.shape[0]

    def flat_idx(res):
        idx = (pos * (res - 1)).astype(jnp.int32)
        return ((jnp.mod(idx[:, 0] - 1, res) * res
                 + jnp.mod(idx[:, 1], res)) * res
                + jnp.mod(idx[:, 2] + 1, res))

    i1 = flat_idx(64)
    i2 = flat_idx(32)
    i3 = flat_idx(16)
    idx0 = (pos * 127).astype(jnp.int32)
    r0xy = (jnp.mod(idx0[:, 0] - 1, 128) * 128 + jnp.mod(idx0[:, 1], 128))
    z0 = jnp.mod(idx0[:, 2] + 1, 128)
    idx3u = (pos * 15).astype(jnp.int32)
    sp3 = ((jnp.mod(idx3u[:, 0], 16) * 16
            + jnp.mod(idx3u[:, 1], 16)) * 16
           + jnp.mod(idx3u[:, 2], 16))
    t_idx = (t * 127).astype(jnp.int32)
    tmod = [jnp.mod(t_idx + k, 64) for k in (-1, 0, 1)]
    sp = jnp.stack([r0xy, z0 // 8, sp3] + [tm // 8 for tm in tmod])

    c0, c1, c2, c3 = (table0.shape[3], table1.shape[3], table2.shape[3],
                      table3.shape[3])
    o1 = 3 * c0
    o2 = o1 + 3 * c1
    o3 = o2 + 3 * c2
    ot = o3 + 3 * c3
    a0 = _fold3(rms_scale[:o1].reshape(3, c0), W_seq[:o1].reshape(3, c0, 64))
    a1 = _fold3(rms_scale[o1:o2].reshape(3, c1),
                W_seq[o1:o2].reshape(3, c1, 64))
    a2 = _fold3(rms_scale[o2:o3].reshape(3, c2),
                W_seq[o2:o3].reshape(3, c2, 64))
    a3 = _fold3(rms_scale[o3:ot].reshape(3, c3),
                W_seq[o3:ot].reshape(3, c3, 64))
    wt = W_seq[ot:]
    st = rms_scale[ot:].reshape(1, 192)

    g1, g2, g3 = _sc_gather(i1, i2, i3, table1.reshape(-1, c1),
                            table2.reshape(-1, c2), table3.reshape(-1, c3))

    t0v = table0.reshape(128 * 128, 128, c0)
    ttv = ts_table1.reshape(16 * 16 * 16, 64, 64)
    tail = jnp.concatenate(
        [dir, t[:, None], (z0 % 8)[:, None].astype(jnp.float32)]
        + [(tm % 8)[:, None].astype(jnp.float32) for tm in tmod], axis=1)

    nsteps = b // _P
    grid_spec = pltpu.PrefetchScalarGridSpec(
        num_scalar_prefetch=1,
        grid=(nsteps,),
        in_specs=(
            [pl.BlockSpec((_P, c1), lambda g, i: (g, 0)),
             pl.BlockSpec((_P, c2), lambda g, i: (g, 0)),
             pl.BlockSpec((_P, c3), lambda g, i: (g, 0))]
            + [pl.BlockSpec((1, 8, c0), functools.partial(_t0map, j))
               for j in range(_P)]
            + [pl.BlockSpec((1, 8, 64), functools.partial(_ttmap, j))
               for j in range(3 * _P)]
            + [pl.BlockSpec((c0, 64), lambda g, i: (0, 0)),
               pl.BlockSpec((c1, 64), lambda g, i: (0, 0)),
               pl.BlockSpec((c2, 64), lambda g, i: (0, 0)),
               pl.BlockSpec((c3, 64), lambda g, i: (0, 0)),
               pl.BlockSpec((1, 192), lambda g, i: (0, 0)),
               pl.BlockSpec((192, 64), lambda g, i: (0, 0)),
               pl.BlockSpec((_P, 8), lambda g, i: (g, 0)),
               pl.BlockSpec((68, 4), lambda g, i: (0, 0)),
               pl.BlockSpec((1, 4), lambda g, i: (0, 0))]),
        out_specs=pl.BlockSpec((_P, 4), lambda g, i: (g, 0)),
    )
    return pl.pallas_call(
        functools.partial(_main_body, d_feat),
        grid_spec=grid_spec,
        out_shape=jax.ShapeDtypeStruct((b, 4), jnp.float32),
        compiler_params=pltpu.CompilerParams(
            dimension_semantics=("arbitrary",)),
    )(sp, g1, g2, g3, *([t0v] * _P), *([ttv] * (3 * _P)), a0, a1, a2, a3,
      st, wt, tail, W_fin, b_fin.reshape(1, 4))


# speedup vs baseline: 11.6434x; 1.7164x over previous
import functools

import jax
import jax.numpy as jnp
from jax import lax
from jax.experimental import pallas as pl
from jax.experimental.pallas import tpu as pltpu
from jax.experimental.pallas import tpu_sc as plsc

_P = 8


def _fold3_body(s_ref, w_ref, o_ref):
    s = s_ref[...]
    w = w_ref[...]
    o_ref[...] = (s[0][:, None] * w[0] + s[1][:, None] * w[1]
                  + s[2][:, None] * w[2])


def _fold3(s3, w3):
    c = w3.shape[1]
    row_tile = min(c, 2048)
    return pl.pallas_call(
        _fold3_body,
        grid=(c // row_tile,),
        in_specs=[
            pl.BlockSpec((3, row_tile), lambda g: (0, g)),
            pl.BlockSpec((3, row_tile, 64), lambda g: (0, g, 0)),
        ],
        out_specs=pl.BlockSpec((row_tile, 64), lambda g: (g, 0)),
        out_shape=jax.ShapeDtypeStruct((c, 64), jnp.float32),
    )(s3, w3)


def _sc_gather(i1, i2, i3, t1, t2, t3):
    b = i1.shape[0]
    info = plsc.get_sparse_core_info()
    nc, ns = info.num_cores, info.num_subcores
    nw = nc * ns
    bpw = b // nw
    c1, c2, c3 = t1.shape[1], t2.shape[1], t3.shape[1]
    ch2 = 8
    ch3 = 8
    cq3 = c3 // 4
    f32 = jnp.float32

    def body(i1_h, i2_h, i3_h, t1_h, t2_h, t3_h, o1_h, o2_h, o3_h,
             idx_v, r1_v, r2_v, r3_v, sem):
        wid = lax.axis_index("s") * nc + lax.axis_index("c")
        base = wid * bpw
        pltpu.sync_copy(i1_h.at[pl.ds(base, bpw)], idx_v)
        pltpu.async_copy(t1_h.at[idx_v], r1_v, sem).wait()
        pltpu.sync_copy(r1_v, o1_h.at[pl.ds(base, bpw)])
        pltpu.sync_copy(i2_h.at[pl.ds(base, bpw)], idx_v)
        for ch in range(bpw // ch2):
            pltpu.async_copy(
                t2_h.at[idx_v.at[pl.ds(ch * ch2, ch2)]], r2_v, sem).wait()
            pltpu.sync_copy(r2_v, o2_h.at[pl.ds(base + ch * ch2, ch2)])
        pltpu.sync_copy(i3_h.at[pl.ds(base, bpw)], idx_v)
        for ch in range(bpw // ch3):
            for q in range(4):
                pltpu.async_copy(
                    t3_h.at[idx_v.at[pl.ds(ch * ch3, ch3)],
                            pl.ds(q * cq3, cq3)],
                    r3_v, sem).wait()
                pltpu.sync_copy(
                    r3_v,
                    o3_h.at[pl.ds(base + ch * ch3, ch3), pl.ds(q * cq3, cq3)])

    out_type = (jax.ShapeDtypeStruct((b, c1), f32),
                jax.ShapeDtypeStruct((b, c2), f32),
                jax.ShapeDtypeStruct((b, c3), f32))
    mesh = plsc.VectorSubcoreMesh(core_axis_name="c", subcore_axis_name="s")
    return pl.kernel(
        body, out_type=out_type, mesh=mesh,
        scratch_types=[pltpu.VMEM((bpw,), jnp.int32),
                       pltpu.VMEM((bpw, c1), f32),
                       pltpu.VMEM((ch2, c2), f32),
                       pltpu.VMEM((ch3, cq3), f32),
                       pltpu.SemaphoreType.DMA],
    )(i1, i2, i3, t1, t2, t3)


def _sel8(refs, rem):
    rows = jnp.stack([r[0] for r in refs])
    iota = lax.broadcasted_iota(jnp.int32, (1, 8), 1).astype(jnp.float32)
    m = jnp.where(rem == iota, 1.0, 0.0)
    return jnp.sum(rows * m[:, :, None], axis=1)


def _main_body(d_feat, *refs):
    sp_ref = refs[0]
    g1_ref, g2_ref, g3_ref = refs[1:4]
    t0b = refs[4:4 + _P]
    ttb = refs[4 + _P:4 + _P + 3 * _P]
    (a0_ref, a1_ref, a2_ref, a3_ref, st_ref, wt_ref, tail_ref, wf_ref,
     bf_ref, out_ref) = refs[4 + 4 * _P:]

    tl = tail_ref[...]
    r3 = jnp.maximum(g3_ref[...], 0.0)
    acc = jnp.dot(r3, a3_ref[...], preferred_element_type=jnp.float32)
    ss_sp = jnp.sum(r3 * r3, axis=1, keepdims=True)
    slabs = jnp.stack([r[0] for r in t0b])
    iota_z = lax.broadcasted_iota(jnp.int32, (1, 128), 1).astype(jnp.float32)
    mz = jnp.where(tl[:, 4:5] == iota_z, 1.0, 0.0)
    r0 = jnp.sum(slabs * mz[:, None, :], axis=2)
    r0 = jnp.maximum(r0, 0.0)
    acc += jnp.dot(r0, a0_ref[...], preferred_element_type=jnp.float32)
    ss_sp += jnp.sum(r0 * r0, axis=1, keepdims=True)
    r1 = jnp.maximum(g1_ref[...], 0.0)
    acc += jnp.dot(r1, a1_ref[...], preferred_element_type=jnp.float32)
    ss_sp += jnp.sum(r1 * r1, axis=1, keepdims=True)
    r2 = jnp.maximum(g2_ref[...], 0.0)
    acc += jnp.dot(r2, a2_ref[...], preferred_element_type=jnp.float32)
    ss_sp += jnp.sum(r2 * r2, axis=1, keepdims=True)
    rt = jnp.maximum(jnp.concatenate(
        [_sel8(ttb[k::3], tl[:, 5 + k:6 + k]) for k in range(3)], axis=1),
        0.0)
    acc += jnp.dot(rt * st_ref[...], wt_ref[...],
                   preferred_element_type=jnp.float32)
    ss = 3.0 * ss_sp + jnp.sum(rt * rt, axis=1, keepdims=True)
    denom = jnp.sqrt(ss) * (float(d_feat) ** -0.5) + 1e-8
    d = acc / denom
    wf = wf_ref[...]
    out_ref[...] = (jnp.dot(d, wf[:64], preferred_element_type=jnp.float32)
                    + jnp.dot(tl[:, :4], wf[64:],
                              preferred_element_type=jnp.float32)
                    + bf_ref[...])


def _t0map(j, g, sp):
    return (sp[0, g * _P + j], 0, 0)


def _ttmap(j, g, sp):
    return (sp[1, g * _P + j // 3], sp[2 + j % 3, g * _P + j // 3], 0)


def kernel(pos, dir, t, table0, table1, table2, table3, ts_table1, rms_scale,
           W_seq, W_fin, b_fin):
    b = pos.shape[0]
    d_feat = rms_scale.shape[0]

    def flat_idx(res):
        idx = (pos * (res - 1)).astype(jnp.int32)
        return ((jnp.mod(idx[:, 0] - 1, res) * res
                 + jnp.mod(idx[:, 1], res)) * res
                + jnp.mod(idx[:, 2] + 1, res))

    i1 = flat_idx(64)
    i2 = flat_idx(32)
    i3 = flat_idx(16)
    idx0 = (pos * 127).astype(jnp.int32)
    r0xy = (jnp.mod(idx0[:, 0] - 1, 128) * 128 + jnp.mod(idx0[:, 1], 128))
    z0 = jnp.mod(idx0[:, 2] + 1, 128)
    idx3u = (pos * 15).astype(jnp.int32)
    sp3 = ((jnp.mod(idx3u[:, 0], 16) * 16
            + jnp.mod(idx3u[:, 1], 16)) * 16
           + jnp.mod(idx3u[:, 2], 16))
    t_idx = (t * 127).astype(jnp.int32)
    tmod = [jnp.mod(t_idx + k, 64) for k in (-1, 0, 1)]
    sp = jnp.stack([r0xy, sp3] + [tm // 8 for tm in tmod])

    c0, c1, c2, c3 = (table0.shape[3], table1.shape[3], table2.shape[3],
                      table3.shape[3])
    o1 = 3 * c0
    o2 = o1 + 3 * c1
    o3 = o2 + 3 * c2
    ot = o3 + 3 * c3
    a0 = _fold3(rms_scale[:o1].reshape(3, c0), W_seq[:o1].reshape(3, c0, 64))
    a1 = _fold3(rms_scale[o1:o2].reshape(3, c1),
                W_seq[o1:o2].reshape(3, c1, 64))
    a2 = _fold3(rms_scale[o2:o3].reshape(3, c2),
                W_seq[o2:o3].reshape(3, c2, 64))
    a3 = _fold3(rms_scale[o3:ot].reshape(3, c3),
                W_seq[o3:ot].reshape(3, c3, 64))
    wt = W_seq[ot:]
    st = rms_scale[ot:].reshape(1, 192)

    g1, g2, g3 = _sc_gather(i1, i2, i3, table1.reshape(-1, c1),
                            table2.reshape(-1, c2), table3.reshape(-1, c3))

    t0v = jnp.transpose(table0, (0, 1, 3, 2)).reshape(128 * 128, c0, 128)
    ttv = ts_table1.reshape(16 * 16 * 16, 64, 64)
    tail = jnp.concatenate(
        [dir, t[:, None], z0[:, None].astype(jnp.float32)]
        + [(tm % 8)[:, None].astype(jnp.float32) for tm in tmod], axis=1)

    nsteps = b // _P
    grid_spec = pltpu.PrefetchScalarGridSpec(
        num_scalar_prefetch=1,
        grid=(nsteps,),
        in_specs=(
            [pl.BlockSpec((_P, c1), lambda g, i: (g, 0)),
             pl.BlockSpec((_P, c2), lambda g, i: (g, 0)),
             pl.BlockSpec((_P, c3), lambda g, i: (g, 0))]
            + [pl.BlockSpec((1, c0, 128), functools.partial(_t0map, j))
               for j in range(_P)]
            + [pl.BlockSpec((1, 8, 64), functools.partial(_ttmap, j))
               for j in range(3 * _P)]
            + [pl.BlockSpec((c0, 64), lambda g, i: (0, 0)),
               pl.BlockSpec((c1, 64), lambda g, i: (0, 0)),
               pl.BlockSpec((c2, 64), lambda g, i: (0, 0)),
               pl.BlockSpec((c3, 64), lambda g, i: (0, 0)),
               pl.BlockSpec((1, 192), lambda g, i: (0, 0)),
               pl.BlockSpec((192, 64), lambda g, i: (0, 0)),
               pl.BlockSpec((_P, 8), lambda g, i: (g, 0)),
               pl.BlockSpec((68, 4), lambda g, i: (0, 0)),
               pl.BlockSpec((1, 4), lambda g, i: (0, 0))]),
        out_specs=pl.BlockSpec((_P, 4), lambda g, i: (g, 0)),
    )
    return pl.pallas_call(
        functools.partial(_main_body, d_feat),
        grid_spec=grid_spec,
        out_shape=jax.ShapeDtypeStruct((b, 4), jnp.float32),
        compiler_params=pltpu.CompilerParams(
            dimension_semantics=("arbitrary",)),
    )(sp, g1, g2, g3, *([t0v] * _P), *([ttv] * (3 * _P)), a0, a1, a2, a3,
      st, wt, tail, W_fin, b_fin.reshape(1, 4))
